# baseline (speedup 1.0000x reference)
import jax
import jax.numpy as jnp
from jax.experimental import pallas as pl

B = 16
T = 64
INPUT_DIM = 1024
OUTPUT_DIM = 263
NB_CODE = 1024
CODE_DIM = 512
WIDTH = 512
NUM_Q = 2


def _dot(x2d, w2d):
    return jax.lax.dot_general(
        x2d.astype(jnp.bfloat16), w2d.astype(jnp.bfloat16),
        (((1,), (0,)), ((), ())),
        preferred_element_type=jnp.float32)


def _shift_r(x, d):
    b, t, c = x.shape
    if d >= t:
        return jnp.zeros_like(x)
    return jnp.concatenate(
        [jnp.zeros((b, d, c), x.dtype), x[:, :t - d, :]], axis=1)


def _shift_l(x, d):
    b, t, c = x.shape
    if d >= t:
        return jnp.zeros_like(x)
    return jnp.concatenate(
        [x[:, d:, :], jnp.zeros((b, d, c), x.dtype)], axis=1)


def _conv3(x, wk, b, dil):
    bb, t, c = x.shape
    xs = jnp.concatenate([_shift_r(x, dil), x, _shift_l(x, dil)], axis=-1)
    y = _dot(xs.reshape(bb * t, 3 * c), wk)
    return y.reshape(bb, t, -1) + b[None]


def _conv1(x, w, b):
    bb, t, c = x.shape
    y = _dot(x.reshape(bb * t, c), w)
    return y.reshape(bb, t, -1) + b[None]


def _resblock(h, wk1, b1, w2, b2, dil):
    o = jnp.maximum(h, 0.0)
    o = _conv3(o, wk1, b1, dil)
    o = jnp.maximum(o, 0.0)
    o = _conv1(o, w2, b2)
    return h + o


def _down(x, wk4, b):
    bb, t, c = x.shape
    x4 = x.reshape(bb, t // 2, 2, c)
    xe = x4[:, :, 0, :]
    xo = x4[:, :, 1, :]
    xs = jnp.concatenate([_shift_r(xo, 1), xe, xo, _shift_l(xe, 1)], axis=-1)
    y = _dot(xs.reshape(bb * (t // 2), 4 * c), wk4)
    return y.reshape(bb, t // 2, -1) + b[None]


def _up(h, wk, b):
    bb, t, c = h.shape
    se = jnp.concatenate([_shift_r(h, 1), h, h], axis=-1)
    so = jnp.concatenate([h, h, _shift_l(h, 1)], axis=-1)
    ye = _dot(se.reshape(bb * t, 3 * c), wk).reshape(bb, t, -1) + b[None]
    yo = _dot(so.reshape(bb * t, 3 * c), wk).reshape(bb, t, -1) + b[None]
    return jnp.concatenate([ye[:, :, None, :], yo[:, :, None, :]],
                           axis=2).reshape(bb, 2 * t, ye.shape[-1])


def _enc0_body(x_ref, wi, bi, wd, bd,
               rw10, rb10, rw20, rb20,
               rw11, rb11, rw21, rb21,
               rw12, rb12, rw22, rb22, o_ref):
    h = jnp.maximum(_conv3(x_ref[...], wi[...], bi[...], 1), 0.0)
    h = _down(h, wd[...], bd[...])
    h = _resblock(h, rw10[...], rb10[...], rw20[...], rb20[...], 1)
    h = _resblock(h, rw11[...], rb11[...], rw21[...], rb21[...], 3)
    h = _resblock(h, rw12[...], rb12[...], rw22[...], rb22[...], 9)
    o_ref[...] = h


def _enc_mid_body(x_ref, wd, bd,
                  rw10, rb10, rw20, rb20,
                  rw11, rb11, rw21, rb21,
                  rw12, rb12, rw22, rb22, o_ref):
    h = _down(x_ref[...], wd[...], bd[...])
    h = _resblock(h, rw10[...], rb10[...], rw20[...], rb20[...], 1)
    h = _resblock(h, rw11[...], rb11[...], rw21[...], rb21[...], 3)
    h = _resblock(h, rw12[...], rb12[...], rw22[...], rb22[...], 9)
    o_ref[...] = h


def _enc_last_body(x_ref, wd, bd,
                   rw10, rb10, rw20, rb20,
                   rw11, rb11, rw21, rb21,
                   rw12, rb12, rw22, rb22,
                   wo, bo, o_ref):
    h = _down(x_ref[...], wd[...], bd[...])
    h = _resblock(h, rw10[...], rb10[...], rw20[...], rb20[...], 1)
    h = _resblock(h, rw11[...], rb11[...], rw21[...], rb21[...], 3)
    h = _resblock(h, rw12[...], rb12[...], rw22[...], rb22[...], 9)
    h = _conv3(h, wo[...], bo[...], 1)
    o_ref[...] = h


def _vq_q_body(res_ref, rowsq_ref, cbt_ref, cbsq_ref, s1_ref, s2_ref, s3_ref,
               prevq_ref, prevc_ref, prevp_ref,
               qout_ref, resn_ref, idx_ref, c_ref, p_ref):
    res = res_ref[...]
    n = res.shape[0]
    code_iota = jax.lax.broadcasted_iota(jnp.int32, (n, NB_CODE), 1)
    dist = (rowsq_ref[...] - 2.0 * _dot(res, cbt_ref[...])) + cbsq_ref[...]
    mind = jnp.min(dist, axis=1, keepdims=True)
    idx = jnp.min(jnp.where(dist <= mind, code_iota, NB_CODE),
                  axis=1).astype(jnp.int32)
    onehot = (idx[:, None] == code_iota).astype(jnp.float32)
    oh_b = onehot.astype(jnp.bfloat16)
    xd = jnp.float32(0.0)
    for s_ref in (s1_ref, s2_ref, s3_ref):
        xd = xd + jax.lax.dot_general(
            oh_b, s_ref[...], (((1,), (0,)), ((), ())),
            preferred_element_type=jnp.float32)
    commit = jnp.mean((res - xd) ** 2)
    probs = jnp.mean(onehot, axis=0)
    perp = jnp.exp(-jnp.sum(probs * jnp.log(probs + 1e-10)))
    xq = res + (xd - res)
    qout_ref[...] = prevq_ref[...] + xq
    resn_ref[...] = res - xq
    idx_ref[...] = idx[None, :]
    c_ref[...] = jnp.reshape(prevc_ref[0, 0] + commit, (1, 1))
    p_ref[...] = jnp.reshape(prevp_ref[0, 0] + perp, (1, 1))


def _dec0_body(x_ref, wi, bi,
               rw10, rb10, rw20, rb20,
               rw11, rb11, rw21, rb21,
               rw12, rb12, rw22, rb22,
               wu, bu, o_ref):
    h = jnp.maximum(_conv3(x_ref[...], wi[...], bi[...], 1), 0.0)
    h = _resblock(h, rw10[...], rb10[...], rw20[...], rb20[...], 9)
    h = _resblock(h, rw11[...], rb11[...], rw21[...], rb21[...], 3)
    h = _resblock(h, rw12[...], rb12[...], rw22[...], rb22[...], 1)
    h = _up(h, wu[...], bu[...])
    o_ref[...] = h


def _dec_mid_body(x_ref,
                  rw10, rb10, rw20, rb20,
                  rw11, rb11, rw21, rb21,
                  rw12, rb12, rw22, rb22,
                  wu, bu, o_ref):
    h = x_ref[...]
    h = _resblock(h, rw10[...], rb10[...], rw20[...], rb20[...], 9)
    h = _resblock(h, rw11[...], rb11[...], rw21[...], rb21[...], 3)
    h = _resblock(h, rw12[...], rb12[...], rw22[...], rb22[...], 1)
    h = _up(h, wu[...], bu[...])
    o_ref[...] = h


def _dec_last_body(x_ref,
                   rw10, rb10, rw20, rb20,
                   rw11, rb11, rw21, rb21,
                   rw12, rb12, rw22, rb22,
                   wu, bu, wm, bm, wo, bo, o_ref):
    h = x_ref[...]
    h = _resblock(h, rw10[...], rb10[...], rw20[...], rb20[...], 9)
    h = _resblock(h, rw11[...], rb11[...], rw21[...], rb21[...], 3)
    h = _resblock(h, rw12[...], rb12[...], rw22[...], rb22[...], 1)
    h = _up(h, wu[...], bu[...])
    h = jnp.maximum(_conv3(h, wm[...], bm[...], 1), 0.0)
    h = _conv3(h, wo[...], bo[...], 1)
    o_ref[...] = h


def _call(body, args, out_shapes):
    return pl.pallas_call(body, out_shape=out_shapes)(*args)


def _imk(w):
    o, i, k = w.shape
    return jnp.transpose(w, (2, 1, 0)).reshape(k * i, o)


def _w1(w):
    return w[:, :, 0].T


def _tb(b):
    return b.reshape(1, -1)


def _res_args(p, pre):
    return [_imk(p[pre + '_w1']), _tb(p[pre + '_b1']),
            _w1(p[pre + '_w2']), _tb(p[pre + '_b2'])]


def kernel(x, params):
    p = params
    f32 = jnp.float32
    x = x.astype(f32)

    enc0_args = ([x, _imk(p['enc_in_w']), _tb(p['enc_in_b']),
                  _imk(p['enc_down0_w']), _tb(p['enc_down0_b'])]
                 + _res_args(p, 'enc_res0_0')
                 + _res_args(p, 'enc_res0_1')
                 + _res_args(p, 'enc_res0_2'))
    h = _call(_enc0_body, enc0_args,
              jax.ShapeDtypeStruct((B, T // 2, WIDTH), f32))

    enc1_args = ([h, _imk(p['enc_down1_w']), _tb(p['enc_down1_b'])]
                 + _res_args(p, 'enc_res1_0')
                 + _res_args(p, 'enc_res1_1')
                 + _res_args(p, 'enc_res1_2'))
    h = _call(_enc_mid_body, enc1_args,
              jax.ShapeDtypeStruct((B, T // 4, WIDTH), f32))

    enc2_args = ([h, _imk(p['enc_down2_w']), _tb(p['enc_down2_b'])]
                 + _res_args(p, 'enc_res2_0')
                 + _res_args(p, 'enc_res2_1')
                 + _res_args(p, 'enc_res2_2')
                 + [_imk(p['enc_out_w']), _tb(p['enc_out_b'])])
    h = _call(_enc_last_body, enc2_args,
              jax.ShapeDtypeStruct((B, T // 8, CODE_DIM), f32))

    tq = T // 8
    flat = h.reshape(B * tq, CODE_DIM)
    cb = p['codebooks']
    cbt = jnp.transpose(cb, (0, 2, 1))
    cbsq = jnp.stack([jnp.sum(cb[q] ** 2, axis=1)[None, :]
                      for q in range(NUM_Q)])
    s1f = jax.lax.reduce_precision(cb, 8, 7)
    r1 = cb - s1f
    s2f = jax.lax.reduce_precision(r1, 8, 7)
    s3f = jax.lax.reduce_precision(r1 - s2f, 8, 7)
    cbs1 = s1f.astype(jnp.bfloat16)
    cbs2 = s2f.astype(jnp.bfloat16)
    cbs3 = s3f.astype(jnp.bfloat16)

    n = B * tq
    vq_out = (jax.ShapeDtypeStruct((n, CODE_DIM), f32),
              jax.ShapeDtypeStruct((n, CODE_DIM), f32),
              jax.ShapeDtypeStruct((1, n), jnp.int32),
              jax.ShapeDtypeStruct((1, 1), f32),
              jax.ShapeDtypeStruct((1, 1), f32))
    zero_q = jnp.zeros((n, CODE_DIM), f32)
    zero_s = jnp.zeros((1, 1), f32)
    res_cur, qout = flat, zero_q
    commit, perp = zero_s, zero_s
    idx_list = []
    for q in range(NUM_Q):
        rowsq = jnp.sum(res_cur ** 2, axis=1, keepdims=True)
        qout, res_cur, idx_q, commit, perp = _call(
            _vq_q_body,
            [res_cur, rowsq, cbt[q], cbsq[q], cbs1[q], cbs2[q], cbs3[q],
             qout, commit, perp],
            vq_out)
        idx_list.append(idx_q)
    idx = jnp.concatenate(idx_list, axis=0)
    perp = perp / NUM_Q

    hq = qout.reshape(B, tq, CODE_DIM)
    dec0_args = ([hq, _imk(p['dec_in_w']), _tb(p['dec_in_b'])]
                 + _res_args(p, 'dec_res0_0')
                 + _res_args(p, 'dec_res0_1')
                 + _res_args(p, 'dec_res0_2')
                 + [_imk(p['dec_up0_w']), _tb(p['dec_up0_b'])])
    h = _call(_dec0_body, dec0_args,
              jax.ShapeDtypeStruct((B, T // 4, WIDTH), f32))

    dec1_args = ([h]
                 + _res_args(p, 'dec_res1_0')
                 + _res_args(p, 'dec_res1_1')
                 + _res_args(p, 'dec_res1_2')
                 + [_imk(p['dec_up1_w']), _tb(p['dec_up1_b'])])
    h = _call(_dec_mid_body, dec1_args,
              jax.ShapeDtypeStruct((B, T // 2, WIDTH), f32))

    dec2_args = ([h]
                 + _res_args(p, 'dec_res2_0')
                 + _res_args(p, 'dec_res2_1')
                 + _res_args(p, 'dec_res2_2')
                 + [_imk(p['dec_up2_w']), _tb(p['dec_up2_b']),
                    _imk(p['dec_mid_w']), _tb(p['dec_mid_b']),
                    _imk(p['dec_out_w']), _tb(p['dec_out_b'])])
    y = _call(_dec_last_body, dec2_args,
              jax.ShapeDtypeStruct((B, T, OUTPUT_DIM), f32))

    return y, idx, commit[0, 0], perp[0, 0]


# speedup vs baseline: 1.1911x; 1.1911x over previous
import jax
import jax.numpy as jnp
from jax.experimental import pallas as pl

B = 16
T = 64
INPUT_DIM = 1024
OUTPUT_DIM = 263
NB_CODE = 1024
CODE_DIM = 512
WIDTH = 512
NUM_Q = 2


def _dot(x2d, w2d):
    return jax.lax.dot_general(
        x2d.astype(jnp.bfloat16), w2d.astype(jnp.bfloat16),
        (((1,), (0,)), ((), ())),
        preferred_element_type=jnp.float32)


def _shift_r(x, d):
    b, t, c = x.shape
    if d >= t:
        return jnp.zeros_like(x)
    return jnp.concatenate(
        [jnp.zeros((b, d, c), x.dtype), x[:, :t - d, :]], axis=1)


def _shift_l(x, d):
    b, t, c = x.shape
    if d >= t:
        return jnp.zeros_like(x)
    return jnp.concatenate(
        [x[:, d:, :], jnp.zeros((b, d, c), x.dtype)], axis=1)


def _conv3(x, wk, b, dil):
    bb, t, c = x.shape
    xs = jnp.concatenate([_shift_r(x, dil), x, _shift_l(x, dil)], axis=-1)
    y = _dot(xs.reshape(bb * t, 3 * c), wk)
    return y.reshape(bb, t, -1) + b[None]


def _conv1(x, w, b):
    bb, t, c = x.shape
    y = _dot(x.reshape(bb * t, c), w)
    return y.reshape(bb, t, -1) + b[None]


def _resblock(h, wk1, b1, w2, b2, dil):
    o = jnp.maximum(h, 0.0)
    o = _conv3(o, wk1, b1, dil)
    o = jnp.maximum(o, 0.0)
    o = _conv1(o, w2, b2)
    return h + o


def _down(x, wk4, b):
    bb, t, c = x.shape
    x4 = x.reshape(bb, t // 2, 2, c)
    xe = x4[:, :, 0, :]
    xo = x4[:, :, 1, :]
    xs = jnp.concatenate([_shift_r(xo, 1), xe, xo, _shift_l(xe, 1)], axis=-1)
    y = _dot(xs.reshape(bb * (t // 2), 4 * c), wk4)
    return y.reshape(bb, t // 2, -1) + b[None]


def _up(h, wk, b):
    bb, t, c = h.shape
    se = jnp.concatenate([_shift_r(h, 1), h, h], axis=-1)
    so = jnp.concatenate([h, h, _shift_l(h, 1)], axis=-1)
    ye = _dot(se.reshape(bb * t, 3 * c), wk).reshape(bb, t, -1) + b[None]
    yo = _dot(so.reshape(bb * t, 3 * c), wk).reshape(bb, t, -1) + b[None]
    return jnp.concatenate([ye[:, :, None, :], yo[:, :, None, :]],
                           axis=2).reshape(bb, 2 * t, ye.shape[-1])


def _enc0_body(x_ref, wi, bi, wd, bd,
               rw10, rb10, rw20, rb20,
               rw11, rb11, rw21, rb21,
               rw12, rb12, rw22, rb22, o_ref):
    h = jnp.maximum(_conv3(x_ref[...], wi[...], bi[...], 1), 0.0)
    h = _down(h, wd[...], bd[...])
    h = _resblock(h, rw10[...], rb10[...], rw20[...], rb20[...], 1)
    h = _resblock(h, rw11[...], rb11[...], rw21[...], rb21[...], 3)
    h = _resblock(h, rw12[...], rb12[...], rw22[...], rb22[...], 9)
    o_ref[...] = h


def _enc_mid_body(x_ref, wd, bd,
                  rw10, rb10, rw20, rb20,
                  rw11, rb11, rw21, rb21,
                  rw12, rb12, rw22, rb22, o_ref):
    h = _down(x_ref[...], wd[...], bd[...])
    h = _resblock(h, rw10[...], rb10[...], rw20[...], rb20[...], 1)
    h = _resblock(h, rw11[...], rb11[...], rw21[...], rb21[...], 3)
    h = _resblock(h, rw12[...], rb12[...], rw22[...], rb22[...], 9)
    o_ref[...] = h


def _enc_last_body(x_ref, wd, bd,
                   rw10, rb10, rw20, rb20,
                   rw11, rb11, rw21, rb21,
                   rw12, rb12, rw22, rb22,
                   wo, bo, o_ref):
    h = _down(x_ref[...], wd[...], bd[...])
    h = _resblock(h, rw10[...], rb10[...], rw20[...], rb20[...], 1)
    h = _resblock(h, rw11[...], rb11[...], rw21[...], rb21[...], 3)
    h = _resblock(h, rw12[...], rb12[...], rw22[...], rb22[...], 9)
    h = _conv3(h, wo[...], bo[...], 1)
    o_ref[...] = h


def _vq_q_body(res_ref, rowsq_ref, cbt_ref, cbsq_ref, s1_ref, s2_ref, s3_ref,
               prevq_ref, prevc_ref, prevp_ref,
               qout_ref, resn_ref, idx_ref, c_ref, p_ref):
    res = res_ref[...]
    n = res.shape[0]
    code_iota = jax.lax.broadcasted_iota(jnp.int32, (n, NB_CODE), 1)
    dist = (rowsq_ref[...] - 2.0 * _dot(res, cbt_ref[...])) + cbsq_ref[...]
    mind = jnp.min(dist, axis=1, keepdims=True)
    idx = jnp.min(jnp.where(dist <= mind, code_iota, NB_CODE),
                  axis=1).astype(jnp.int32)
    onehot = (idx[:, None] == code_iota).astype(jnp.float32)
    oh_b = onehot.astype(jnp.bfloat16)
    xd = jnp.float32(0.0)
    for s_ref in (s1_ref, s2_ref, s3_ref):
        xd = xd + jax.lax.dot_general(
            oh_b, s_ref[...], (((1,), (0,)), ((), ())),
            preferred_element_type=jnp.float32)
    commit = jnp.mean((res - xd) ** 2)
    probs = jnp.mean(onehot, axis=0)
    perp = jnp.exp(-jnp.sum(probs * jnp.log(probs + 1e-10)))
    xq = res + (xd - res)
    qout_ref[...] = prevq_ref[...] + xq
    resn_ref[...] = res - xq
    idx_ref[...] = idx[None, :]
    c_ref[...] = jnp.reshape(prevc_ref[0, 0] + commit, (1, 1))
    p_ref[...] = jnp.reshape(prevp_ref[0, 0] + perp, (1, 1))


def _dec0_body(x_ref, wi, bi,
               rw10, rb10, rw20, rb20,
               rw11, rb11, rw21, rb21,
               rw12, rb12, rw22, rb22,
               wu, bu, o_ref):
    h = jnp.maximum(_conv3(x_ref[...], wi[...], bi[...], 1), 0.0)
    h = _resblock(h, rw10[...], rb10[...], rw20[...], rb20[...], 9)
    h = _resblock(h, rw11[...], rb11[...], rw21[...], rb21[...], 3)
    h = _resblock(h, rw12[...], rb12[...], rw22[...], rb22[...], 1)
    h = _up(h, wu[...], bu[...])
    o_ref[...] = h


def _dec_mid_body(x_ref,
                  rw10, rb10, rw20, rb20,
                  rw11, rb11, rw21, rb21,
                  rw12, rb12, rw22, rb22,
                  wu, bu, o_ref):
    h = x_ref[...]
    h = _resblock(h, rw10[...], rb10[...], rw20[...], rb20[...], 9)
    h = _resblock(h, rw11[...], rb11[...], rw21[...], rb21[...], 3)
    h = _resblock(h, rw12[...], rb12[...], rw22[...], rb22[...], 1)
    h = _up(h, wu[...], bu[...])
    o_ref[...] = h


def _dec_last_body(x_ref,
                   rw10, rb10, rw20, rb20,
                   rw11, rb11, rw21, rb21,
                   rw12, rb12, rw22, rb22,
                   wu, bu, wm, bm, wo, bo, o_ref):
    h = x_ref[...]
    h = _resblock(h, rw10[...], rb10[...], rw20[...], rb20[...], 9)
    h = _resblock(h, rw11[...], rb11[...], rw21[...], rb21[...], 3)
    h = _resblock(h, rw12[...], rb12[...], rw22[...], rb22[...], 1)
    h = _up(h, wu[...], bu[...])
    h = jnp.maximum(_conv3(h, wm[...], bm[...], 1), 0.0)
    h = _conv3(h, wo[...], bo[...], 1)
    o_ref[...] = h


def _call(body, args, out_shapes):
    return pl.pallas_call(body, out_shape=out_shapes)(*args)


def _imk(w):
    o, i, k = w.shape
    return jnp.transpose(w, (2, 1, 0)).reshape(k * i, o).astype(jnp.bfloat16)


def _w1(w):
    return w[:, :, 0].T.astype(jnp.bfloat16)


def _tb(b):
    return b.reshape(1, -1)


def _res_args(p, pre):
    return [_imk(p[pre + '_w1']), _tb(p[pre + '_b1']),
            _w1(p[pre + '_w2']), _tb(p[pre + '_b2'])]


def kernel(x, params):
    p = params
    f32 = jnp.float32
    x = x.astype(f32)

    enc0_args = ([x, _imk(p['enc_in_w']), _tb(p['enc_in_b']),
                  _imk(p['enc_down0_w']), _tb(p['enc_down0_b'])]
                 + _res_args(p, 'enc_res0_0')
                 + _res_args(p, 'enc_res0_1')
                 + _res_args(p, 'enc_res0_2'))
    h = _call(_enc0_body, enc0_args,
              jax.ShapeDtypeStruct((B, T // 2, WIDTH), f32))

    enc1_args = ([h, _imk(p['enc_down1_w']), _tb(p['enc_down1_b'])]
                 + _res_args(p, 'enc_res1_0')
                 + _res_args(p, 'enc_res1_1')
                 + _res_args(p, 'enc_res1_2'))
    h = _call(_enc_mid_body, enc1_args,
              jax.ShapeDtypeStruct((B, T // 4, WIDTH), f32))

    enc2_args = ([h, _imk(p['enc_down2_w']), _tb(p['enc_down2_b'])]
                 + _res_args(p, 'enc_res2_0')
                 + _res_args(p, 'enc_res2_1')
                 + _res_args(p, 'enc_res2_2')
                 + [_imk(p['enc_out_w']), _tb(p['enc_out_b'])])
    h = _call(_enc_last_body, enc2_args,
              jax.ShapeDtypeStruct((B, T // 8, CODE_DIM), f32))

    tq = T // 8
    flat = h.reshape(B * tq, CODE_DIM)
    cb = p['codebooks']
    cbt = jnp.transpose(cb, (0, 2, 1)).astype(jnp.bfloat16)
    cbsq = jnp.stack([jnp.sum(cb[q] ** 2, axis=1)[None, :]
                      for q in range(NUM_Q)])
    s1f = jax.lax.reduce_precision(cb, 8, 7)
    r1 = cb - s1f
    s2f = jax.lax.reduce_precision(r1, 8, 7)
    s3f = jax.lax.reduce_precision(r1 - s2f, 8, 7)
    cbs1 = s1f.astype(jnp.bfloat16)
    cbs2 = s2f.astype(jnp.bfloat16)
    cbs3 = s3f.astype(jnp.bfloat16)

    n = B * tq
    vq_out = (jax.ShapeDtypeStruct((n, CODE_DIM), f32),
              jax.ShapeDtypeStruct((n, CODE_DIM), f32),
              jax.ShapeDtypeStruct((1, n), jnp.int32),
              jax.ShapeDtypeStruct((1, 1), f32),
              jax.ShapeDtypeStruct((1, 1), f32))
    zero_q = jnp.zeros((n, CODE_DIM), f32)
    zero_s = jnp.zeros((1, 1), f32)
    res_cur, qout = flat, zero_q
    commit, perp = zero_s, zero_s
    idx_list = []
    for q in range(NUM_Q):
        rowsq = jnp.sum(res_cur ** 2, axis=1, keepdims=True)
        qout, res_cur, idx_q, commit, perp = _call(
            _vq_q_body,
            [res_cur, rowsq, cbt[q], cbsq[q], cbs1[q], cbs2[q], cbs3[q],
             qout, commit, perp],
            vq_out)
        idx_list.append(idx_q)
    idx = jnp.concatenate(idx_list, axis=0)
    perp = perp / NUM_Q

    hq = qout.reshape(B, tq, CODE_DIM)
    dec0_args = ([hq, _imk(p['dec_in_w']), _tb(p['dec_in_b'])]
                 + _res_args(p, 'dec_res0_0')
                 + _res_args(p, 'dec_res0_1')
                 + _res_args(p, 'dec_res0_2')
                 + [_imk(p['dec_up0_w']), _tb(p['dec_up0_b'])])
    h = _call(_dec0_body, dec0_args,
              jax.ShapeDtypeStruct((B, T // 4, WIDTH), f32))

    dec1_args = ([h]
                 + _res_args(p, 'dec_res1_0')
                 + _res_args(p, 'dec_res1_1')
                 + _res_args(p, 'dec_res1_2')
                 + [_imk(p['dec_up1_w']), _tb(p['dec_up1_b'])])
    h = _call(_dec_mid_body, dec1_args,
              jax.ShapeDtypeStruct((B, T // 2, WIDTH), f32))

    dec2_args = ([h]
                 + _res_args(p, 'dec_res2_0')
                 + _res_args(p, 'dec_res2_1')
                 + _res_args(p, 'dec_res2_2')
                 + [_imk(p['dec_up2_w']), _tb(p['dec_up2_b']),
                    _imk(p['dec_mid_w']), _tb(p['dec_mid_b']),
                    _imk(p['dec_out_w']), _tb(p['dec_out_b'])])
    y = _call(_dec_last_body, dec2_args,
              jax.ShapeDtypeStruct((B, T, OUTPUT_DIM), f32))

    return y, idx, commit[0, 0], perp[0, 0]


# speedup vs baseline: 1.2083x; 1.0145x over previous
import jax
import jax.numpy as jnp
from jax.experimental import pallas as pl

B = 16
T = 64
INPUT_DIM = 1024
OUTPUT_DIM = 263
NB_CODE = 1024
CODE_DIM = 512
WIDTH = 512
NUM_Q = 2


def _dot(x2d, w2d):
    return jax.lax.dot_general(
        x2d.astype(jnp.bfloat16), w2d.astype(jnp.bfloat16),
        (((1,), (0,)), ((), ())),
        preferred_element_type=jnp.float32)


def _shift_r(x, d):
    b, t, c = x.shape
    if d >= t:
        return jnp.zeros_like(x)
    return jnp.concatenate(
        [jnp.zeros((b, d, c), x.dtype), x[:, :t - d, :]], axis=1)


def _shift_l(x, d):
    b, t, c = x.shape
    if d >= t:
        return jnp.zeros_like(x)
    return jnp.concatenate(
        [x[:, d:, :], jnp.zeros((b, d, c), x.dtype)], axis=1)


def _conv3(x, wk, b, dil):
    bb, t, c = x.shape
    xs = jnp.concatenate([_shift_r(x, dil), x, _shift_l(x, dil)], axis=-1)
    y = _dot(xs.reshape(bb * t, 3 * c), wk)
    return y.reshape(bb, t, -1) + b[None]


def _conv1(x, w, b):
    bb, t, c = x.shape
    y = _dot(x.reshape(bb * t, c), w)
    return y.reshape(bb, t, -1) + b[None]


def _resblock(h, wk1, b1, w2, b2, dil):
    o = jnp.maximum(h, 0.0)
    o = _conv3(o, wk1, b1, dil)
    o = jnp.maximum(o, 0.0)
    o = _conv1(o, w2, b2)
    return h + o


def _down(x, wk4, b):
    bb, t, c = x.shape
    x4 = x.reshape(bb, t // 2, 2, c)
    xe = x4[:, :, 0, :]
    xo = x4[:, :, 1, :]
    xs = jnp.concatenate([_shift_r(xo, 1), xe, xo, _shift_l(xe, 1)], axis=-1)
    y = _dot(xs.reshape(bb * (t // 2), 4 * c), wk4)
    return y.reshape(bb, t // 2, -1) + b[None]


def _up(h, wk, b):
    bb, t, c = h.shape
    se = jnp.concatenate([_shift_r(h, 1), h, h], axis=-1)
    so = jnp.concatenate([h, h, _shift_l(h, 1)], axis=-1)
    ye = _dot(se.reshape(bb * t, 3 * c), wk).reshape(bb, t, -1) + b[None]
    yo = _dot(so.reshape(bb * t, 3 * c), wk).reshape(bb, t, -1) + b[None]
    return jnp.concatenate([ye[:, :, None, :], yo[:, :, None, :]],
                           axis=2).reshape(bb, 2 * t, ye.shape[-1])


def _enc0_body(x_ref, wi, bi, wd, bd,
               rw10, rb10, rw20, rb20,
               rw11, rb11, rw21, rb21,
               rw12, rb12, rw22, rb22, o_ref):
    h = jnp.maximum(_conv3(x_ref[...], wi[...], bi[...], 1), 0.0)
    h = _down(h, wd[...], bd[...])
    h = _resblock(h, rw10[...], rb10[...], rw20[...], rb20[...], 1)
    h = _resblock(h, rw11[...], rb11[...], rw21[...], rb21[...], 3)
    h = _resblock(h, rw12[...], rb12[...], rw22[...], rb22[...], 9)
    o_ref[...] = h


def _enc12_body(x_ref, wd1, bd1,
                a10, a11, a12, a13, b10, b11, b12, b13, c10, c11, c12, c13,
                wd2, bd2,
                a20, a21, a22, a23, b20, b21, b22, b23, c20, c21, c22, c23,
                wo, bo, o_ref):
    h = _down(x_ref[...], wd1[...], bd1[...])
    h = _resblock(h, a10[...], a11[...], a12[...], a13[...], 1)
    h = _resblock(h, b10[...], b11[...], b12[...], b13[...], 3)
    h = _resblock(h, c10[...], c11[...], c12[...], c13[...], 9)
    h = _down(h, wd2[...], bd2[...])
    h = _resblock(h, a20[...], a21[...], a22[...], a23[...], 1)
    h = _resblock(h, b20[...], b21[...], b22[...], b23[...], 3)
    h = _resblock(h, c20[...], c21[...], c22[...], c23[...], 9)
    h = _conv3(h, wo[...], bo[...], 1)
    o_ref[...] = h


def _vq_q_body(res_ref, rowsq_ref, cbt_ref, cbsq_ref, s1_ref, s2_ref, s3_ref,
               prevq_ref, prevc_ref, prevp_ref,
               qout_ref, resn_ref, idx_ref, c_ref, p_ref):
    res = res_ref[...]
    n = res.shape[0]
    code_iota = jax.lax.broadcasted_iota(jnp.int32, (n, NB_CODE), 1)
    dist = (rowsq_ref[...] - 2.0 * _dot(res, cbt_ref[...])) + cbsq_ref[...]
    mind = jnp.min(dist, axis=1, keepdims=True)
    idx = jnp.min(jnp.where(dist <= mind, code_iota, NB_CODE),
                  axis=1).astype(jnp.int32)
    onehot = (idx[:, None] == code_iota).astype(jnp.float32)
    oh_b = onehot.astype(jnp.bfloat16)
    xd = jnp.float32(0.0)
    for s_ref in (s1_ref, s2_ref, s3_ref):
        xd = xd + jax.lax.dot_general(
            oh_b, s_ref[...], (((1,), (0,)), ((), ())),
            preferred_element_type=jnp.float32)
    commit = jnp.mean((res - xd) ** 2)
    probs = jnp.mean(onehot, axis=0)
    perp = jnp.exp(-jnp.sum(probs * jnp.log(probs + 1e-10)))
    xq = res + (xd - res)
    qout_ref[...] = prevq_ref[...] + xq
    resn_ref[...] = res - xq
    idx_ref[...] = idx[None, :]
    c_ref[...] = jnp.reshape(prevc_ref[0, 0] + commit, (1, 1))
    p_ref[...] = jnp.reshape(prevp_ref[0, 0] + perp, (1, 1))


def _dec01_body(x_ref, wi, bi,
                a10, a11, a12, a13, b10, b11, b12, b13, c10, c11, c12, c13,
                wu0, bu0,
                a20, a21, a22, a23, b20, b21, b22, b23, c20, c21, c22, c23,
                wu1, bu1, o_ref):
    h = jnp.maximum(_conv3(x_ref[...], wi[...], bi[...], 1), 0.0)
    h = _resblock(h, a10[...], a11[...], a12[...], a13[...], 9)
    h = _resblock(h, b10[...], b11[...], b12[...], b13[...], 3)
    h = _resblock(h, c10[...], c11[...], c12[...], c13[...], 1)
    h = _up(h, wu0[...], bu0[...])
    h = _resblock(h, a20[...], a21[...], a22[...], a23[...], 9)
    h = _resblock(h, b20[...], b21[...], b22[...], b23[...], 3)
    h = _resblock(h, c20[...], c21[...], c22[...], c23[...], 1)
    h = _up(h, wu1[...], bu1[...])
    o_ref[...] = h


def _dec_last_body(x_ref,
                   rw10, rb10, rw20, rb20,
                   rw11, rb11, rw21, rb21,
                   rw12, rb12, rw22, rb22,
                   wu, bu, wm, bm, wo, bo, o_ref):
    h = x_ref[...]
    h = _resblock(h, rw10[...], rb10[...], rw20[...], rb20[...], 9)
    h = _resblock(h, rw11[...], rb11[...], rw21[...], rb21[...], 3)
    h = _resblock(h, rw12[...], rb12[...], rw22[...], rb22[...], 1)
    h = _up(h, wu[...], bu[...])
    h = jnp.maximum(_conv3(h, wm[...], bm[...], 1), 0.0)
    h = _conv3(h, wo[...], bo[...], 1)
    o_ref[...] = h


def _call(body, args, out_shapes):
    return pl.pallas_call(body, out_shape=out_shapes)(*args)


def _imk(w):
    o, i, k = w.shape
    return jnp.transpose(w, (2, 1, 0)).reshape(k * i, o).astype(jnp.bfloat16)


def _w1(w):
    return w[:, :, 0].T.astype(jnp.bfloat16)


def _tb(b):
    return b.reshape(1, -1)


def _res_args(p, pre):
    return [_imk(p[pre + '_w1']), _tb(p[pre + '_b1']),
            _w1(p[pre + '_w2']), _tb(p[pre + '_b2'])]


def kernel(x, params):
    p = params
    f32 = jnp.float32
    x = x.astype(f32)

    enc0_args = ([x, _imk(p['enc_in_w']), _tb(p['enc_in_b']),
                  _imk(p['enc_down0_w']), _tb(p['enc_down0_b'])]
                 + _res_args(p, 'enc_res0_0')
                 + _res_args(p, 'enc_res0_1')
                 + _res_args(p, 'enc_res0_2'))
    h = _call(_enc0_body, enc0_args,
              jax.ShapeDtypeStruct((B, T // 2, WIDTH), f32))

    enc12_args = ([h, _imk(p['enc_down1_w']), _tb(p['enc_down1_b'])]
                  + _res_args(p, 'enc_res1_0')
                  + _res_args(p, 'enc_res1_1')
                  + _res_args(p, 'enc_res1_2')
                  + [_imk(p['enc_down2_w']), _tb(p['enc_down2_b'])]
                  + _res_args(p, 'enc_res2_0')
                  + _res_args(p, 'enc_res2_1')
                  + _res_args(p, 'enc_res2_2')
                  + [_imk(p['enc_out_w']), _tb(p['enc_out_b'])])
    h = _call(_enc12_body, enc12_args,
              jax.ShapeDtypeStruct((B, T // 8, CODE_DIM), f32))

    tq = T // 8
    flat = h.reshape(B * tq, CODE_DIM)
    cb = p['codebooks']
    cbt = jnp.transpose(cb, (0, 2, 1)).astype(jnp.bfloat16)
    cbsq = jnp.stack([jnp.sum(cb[q] ** 2, axis=1)[None, :]
                      for q in range(NUM_Q)])
    s1f = jax.lax.reduce_precision(cb, 8, 7)
    r1 = cb - s1f
    s2f = jax.lax.reduce_precision(r1, 8, 7)
    s3f = jax.lax.reduce_precision(r1 - s2f, 8, 7)
    cbs1 = s1f.astype(jnp.bfloat16)
    cbs2 = s2f.astype(jnp.bfloat16)
    cbs3 = s3f.astype(jnp.bfloat16)

    n = B * tq
    vq_out = (jax.ShapeDtypeStruct((n, CODE_DIM), f32),
              jax.ShapeDtypeStruct((n, CODE_DIM), f32),
              jax.ShapeDtypeStruct((1, n), jnp.int32),
              jax.ShapeDtypeStruct((1, 1), f32),
              jax.ShapeDtypeStruct((1, 1), f32))
    zero_q = jnp.zeros((n, CODE_DIM), f32)
    zero_s = jnp.zeros((1, 1), f32)
    res_cur, qout = flat, zero_q
    commit, perp = zero_s, zero_s
    idx_list = []
    for q in range(NUM_Q):
        rowsq = jnp.sum(res_cur ** 2, axis=1, keepdims=True)
        qout, res_cur, idx_q, commit, perp = _call(
            _vq_q_body,
            [res_cur, rowsq, cbt[q], cbsq[q], cbs1[q], cbs2[q], cbs3[q],
             qout, commit, perp],
            vq_out)
        idx_list.append(idx_q)
    idx = jnp.concatenate(idx_list, axis=0)
    perp = perp / NUM_Q

    hq = qout.reshape(B, tq, CODE_DIM)
    dec01_args = ([hq, _imk(p['dec_in_w']), _tb(p['dec_in_b'])]
                  + _res_args(p, 'dec_res0_0')
                  + _res_args(p, 'dec_res0_1')
                  + _res_args(p, 'dec_res0_2')
                  + [_imk(p['dec_up0_w']), _tb(p['dec_up0_b'])]
                  + _res_args(p, 'dec_res1_0')
                  + _res_args(p, 'dec_res1_1')
                  + _res_args(p, 'dec_res1_2')
                  + [_imk(p['dec_up1_w']), _tb(p['dec_up1_b'])])
    h = _call(_dec01_body, dec01_args,
              jax.ShapeDtypeStruct((B, T // 2, WIDTH), f32))

    dec2_args = ([h]
                 + _res_args(p, 'dec_res2_0')
                 + _res_args(p, 'dec_res2_1')
                 + _res_args(p, 'dec_res2_2')
                 + [_imk(p['dec_up2_w']), _tb(p['dec_up2_b']),
                    _imk(p['dec_mid_w']), _tb(p['dec_mid_b']),
                    _imk(p['dec_out_w']), _tb(p['dec_out_b'])])
    y = _call(_dec_last_body, dec2_args,
              jax.ShapeDtypeStruct((B, T, OUTPUT_DIM), f32))

    return y, idx, commit[0, 0], perp[0, 0]
